# baseline (speedup 1.0000x reference)
import functools

import jax
import jax.numpy as jnp
from jax import lax
from jax.experimental import pallas as pl
from jax.experimental.pallas import tpu as pltpu
from jax.experimental.pallas import tpu_sc as plsc

D = 64
K = 20
CH = 16
HALF = D // 2
TBI = 512


def _tr_body(x_ref, o_ref):
  x = x_ref[...]
  h = TBI // 2
  o_ref[:, 0:D] = x[:, 0:h].T
  o_ref[:, D:2 * D] = x[:, h:TBI].T


def _pack_table(emb_t):
  v = emb_t.shape[1]
  grid = (v + TBI - 1) // TBI
  return pl.pallas_call(
      _tr_body,
      grid=(grid,),
      in_specs=[pl.BlockSpec((D, TBI), lambda j: (0, j))],
      out_specs=pl.BlockSpec((TBI // 2, 2 * D), lambda j: (j, 0)),
      out_shape=jax.ShapeDtypeStruct((grid * (TBI // 2), 2 * D),
                                     jnp.float32),
  )(emb_t)


def _sc_body(cw_hbm, tw_hbm, ow_hbm, pv_hbm, pu_hbm,
             scores_hbm, sumexp_hbm,
             idxc, idxt, idxo, idxc2, idxt2, idxo2,
             rows_c, rows_t, rows_o,
             nscr, scores_v, sumexp_v, semc, semt, semo,
             *, bpw, nchunk):
  nc = plsc.get_sparse_core_info().num_cores
  wid = lax.axis_index("s") * nc + lax.axis_index("c")
  rowids = lax.iota(jnp.int32, CH)

  base0 = wid * bpw
  pltpu.sync_copy(cw_hbm.at[pl.ds(base0, bpw)], idxc)
  pltpu.sync_copy(tw_hbm.at[pl.ds(base0, bpw)], idxt)
  pltpu.sync_copy(ow_hbm.at[pl.ds(base0 * K, bpw * K)], idxo)

  def make_halve(src, dst):
    def halve(j, carry):
      i = src[pl.ds(j * CH, CH)]
      dst[pl.ds(j * CH, CH)] = ((i & -TBI) >> 1) | (i & (TBI // 2 - 1))
      return carry
    return halve

  lax.fori_loop(0, bpw // CH, make_halve(idxc, idxc2), 0)
  lax.fori_loop(0, bpw // CH, make_halve(idxt, idxt2), 0)
  lax.fori_loop(0, bpw * K // CH, make_halve(idxo, idxo2), 0)

  def issue(chunk, p):
    pltpu.async_copy(pv_hbm.at[idxc2.at[pl.ds(chunk * CH, CH)]],
                     rows_c[p], semc[p])
    pltpu.async_copy(pu_hbm.at[idxt2.at[pl.ds(chunk * CH, CH)]],
                     rows_t[p], semt[p])
    pltpu.async_copy(pu_hbm.at[idxo2.at[pl.ds(chunk * CH * K, CH * K)]],
                     rows_o[p], semo[p])

  def drain(p):
    pltpu.make_async_copy(pv_hbm.at[pl.ds(0, CH)], rows_c[p], semc[p]).wait()
    pltpu.make_async_copy(pu_hbm.at[pl.ds(0, CH)], rows_t[p], semt[p]).wait()
    pltpu.make_async_copy(pu_hbm.at[pl.ds(0, CH * K)], rows_o[p],
                          semo[p]).wait()

  issue(0, 0)

  def pair_body(pair, carry):
    for p in (0, 1):
      c = 2 * pair + p
      drain(p)
      issue(jnp.minimum(c + 1, nchunk - 1), 1 - p)

      parc = (idxc[pl.ds(c * CH, CH)] & (TBI // 2)) >> 2
      part = (idxt[pl.ds(c * CH, CH)] & (TBI // 2)) >> 2

      score = jnp.zeros((CH,), jnp.float32)
      sumexp = jnp.zeros((CH,), jnp.float32)
      for half in (0, 1):
        base_d = half * HALF
        stag = [(rowids + base_d + d) & (D - 1) for d in range(HALF)]
        ccols = [plsc.load_gather(rows_c[p], [rowids, parc + stag[d]])
                 for d in range(HALF)]
        tacc = [jnp.zeros((CH,), jnp.float32) for _ in range(4)]
        for d in range(HALF):
          tv = plsc.load_gather(rows_t[p], [rowids, part + stag[d]])
          tacc[d % 4] = tacc[d % 4] + ccols[d] * tv
        score = score + (tacc[0] + tacc[1]) + (tacc[2] + tacc[3])

        def k_body(k, se, *, _p=p, _c=c, _ccols=ccols, _stag=stag,
                   _half=half):
          orow = rowids * K + k
          paro = (plsc.load_gather(idxo, [_c * CH * K + orow]) & (TBI // 2)) >> 2
          nacc = [jnp.zeros((CH,), jnp.float32) for _ in range(4)]
          for d in range(HALF):
            ov = plsc.load_gather(rows_o[_p], [orow, paro + _stag[d]])
            nacc[d % 4] = nacc[d % 4] + _ccols[d] * ov
          total = (nacc[0] + nacc[1]) + (nacc[2] + nacc[3])
          if _half == 0:
            nscr[pl.ds(k * CH, CH)] = total
            return se
          return se + jnp.exp(nscr[pl.ds(k * CH, CH)] + total)

        sumexp = lax.fori_loop(0, K, k_body, sumexp)

      scores_v[pl.ds(c * CH, CH)] = score
      sumexp_v[pl.ds(c * CH, CH)] = sumexp
    return carry

  lax.fori_loop(0, nchunk // 2, pair_body, 0)
  drain(0)
  pltpu.sync_copy(scores_v, scores_hbm.at[pl.ds(wid * bpw, bpw)])
  pltpu.sync_copy(sumexp_v, sumexp_hbm.at[pl.ds(wid * bpw, bpw)])


def _sc_gather_dots(cw, tw, ow, pv, pu):
  b = cw.shape[0]
  info = plsc.get_sparse_core_info()
  nw = info.num_cores * info.num_subcores
  bpw = b // nw
  nchunk = bpw // CH
  mesh = plsc.VectorSubcoreMesh(core_axis_name="c", subcore_axis_name="s")
  f32 = jnp.float32
  i32 = jnp.int32
  run = pl.kernel(
      functools.partial(_sc_body, bpw=bpw, nchunk=nchunk),
      out_type=(jax.ShapeDtypeStruct((b,), f32),
                jax.ShapeDtypeStruct((b,), f32)),
      mesh=mesh,
      compiler_params=pltpu.CompilerParams(needs_layout_passes=False,
                                           use_tc_tiling_on_sc=True),
      scratch_types=[
          pltpu.VMEM((bpw,), i32),
          pltpu.VMEM((bpw,), i32),
          pltpu.VMEM((bpw * K,), i32),
          pltpu.VMEM((bpw,), i32),
          pltpu.VMEM((bpw,), i32),
          pltpu.VMEM((bpw * K,), i32),
          [pltpu.VMEM((CH, 2 * D), f32)] * 2,
          [pltpu.VMEM((CH, 2 * D), f32)] * 2,
          [pltpu.VMEM((CH * K, 2 * D), f32)] * 2,
          pltpu.VMEM((CH * K,), f32),
          pltpu.VMEM((bpw,), f32),
          pltpu.VMEM((bpw,), f32),
          [pltpu.SemaphoreType.DMA] * 2,
          [pltpu.SemaphoreType.DMA] * 2,
          [pltpu.SemaphoreType.DMA] * 2,
      ],
  )
  return run(cw, tw, ow, pv, pu)


def _finish_body(s_ref, e_ref, o_ref):
  s = s_ref[...]
  e = e_ref[...]
  n = s.size
  o_ref[0, 0] = -(jnp.sum(s) - jnp.sum(jnp.log(e))) / n


def _tc_finish(scores, sumexp):
  b = scores.shape[0]
  rows = b // 128
  out = pl.pallas_call(
      _finish_body,
      out_shape=jax.ShapeDtypeStruct((1, 1), jnp.float32),
      out_specs=pl.BlockSpec(memory_space=pltpu.SMEM),
  )(scores.reshape(rows, 128), sumexp.reshape(rows, 128))
  return out[0, 0]


def kernel(center_words, target_words, outer_words, emb_v, emb_u):
  cw = center_words.reshape(-1).astype(jnp.int32)
  tw = target_words.reshape(-1).astype(jnp.int32)
  ow = outer_words.reshape(-1).astype(jnp.int32)
  pv = _pack_table(emb_v.T)
  pu = _pack_table(emb_u.T)
  scores, sumexp = _sc_gather_dots(cw, tw, ow, pv, pu)
  return _tc_finish(scores, sumexp)
